# baseline (speedup 1.0000x reference)
import functools

import jax
import jax.numpy as jnp
from jax import lax
from jax.experimental import pallas as pl
from jax.experimental.pallas import tpu as pltpu
from jax.experimental.pallas import tpu_sc as plsc

N = 10000
E = 320000
H = 128
G = 128
C = 10

NC = 2
NS = 16
NW = NC * NS
NPAD = 10240
ROWS_PT = NPAD // NS
CHUNK = 128
NCH = E // CHUNK
CPT = 80
NBUF = 2
IBUF = 4


@functools.cache
def _sc_kernels():
    mesh = plsc.VectorSubcoreMesh(core_axis_name="c", subcore_axis_name="s")

    @functools.partial(
        pl.kernel,
        mesh=mesh,
        out_type=jax.ShapeDtypeStruct((NC, NPAD), jnp.float32),
        scratch_types=[
            pltpu.VMEM((CPT * CHUNK,), jnp.int32),
            pltpu.VMEM((NBUF, CHUNK), jnp.int32),
            pltpu.VMEM((CHUNK,), jnp.float32),
            pltpu.VMEM((ROWS_PT,), jnp.float32),
            pltpu.VMEM_SHARED((NPAD,), jnp.float32),
        ] + [pltpu.SemaphoreType.DMA for _ in range(NBUF)],
    )
    def sc_degree(dst_hbm, out_hbm, idxall, idxr, ones_v, zero_v, acc, *ssem):
        c = lax.axis_index("c")
        s = lax.axis_index("s")
        w = c * NS + s

        pltpu.sync_copy(dst_hbm.at[pl.ds(w * CPT * CHUNK, CPT * CHUNK)],
                        idxall)

        for i in range(CHUNK // 16):
            ones_v[pl.ds(i * 16, 16)] = jnp.ones((16,), jnp.float32)

        def zb(i, carry):
            zero_v[pl.ds(i * 16, 16)] = jnp.zeros((16,), jnp.float32)
            return carry

        lax.fori_loop(0, ROWS_PT // 16, zb, 0)
        pltpu.sync_copy(zero_v, acc.at[pl.ds(s * ROWS_PT, ROWS_PT)])
        plsc.subcore_barrier()

        def fire_scatter(j, b):
            for k in range(CHUNK // 16):
                idxr[b, pl.ds(k * 16, 16)] = idxall[
                    pl.ds(j * CHUNK + k * 16, 16)]
            pltpu.async_copy(ones_v, acc.at[idxr.at[b]], ssem[b], add=True)

        for b in range(NBUF):
            fire_scatter(b, b)

        def body(i, carry):
            j0 = i * NBUF
            for b in range(NBUF):
                pltpu.make_async_copy(ones_v, acc.at[idxr.at[b]],
                                      ssem[b]).wait()
                fire_scatter(j0 + b + NBUF, b)
            return carry

        lax.fori_loop(0, CPT // NBUF - 1, body, 0)
        for b in range(NBUF):
            pltpu.make_async_copy(ones_v, acc.at[idxr.at[b]],
                                  ssem[b]).wait()

        plsc.subcore_barrier()
        pltpu.sync_copy(acc.at[pl.ds(s * ROWS_PT, ROWS_PT)],
                        out_hbm.at[c, pl.ds(s * ROWS_PT, ROWS_PT)])

    @functools.partial(
        pl.kernel,
        mesh=mesh,
        out_type=jax.ShapeDtypeStruct((NC, NPAD, H), jnp.float32),
        scratch_types=(
            [
                pltpu.VMEM((IBUF, CHUNK), jnp.int32),
                pltpu.VMEM((IBUF, CHUNK), jnp.int32),
            ]
            + [pltpu.VMEM((CHUNK, H), jnp.float32) for _ in range(NBUF)]
            + [
                pltpu.VMEM((32, H), jnp.float32),
                pltpu.VMEM_SHARED((NPAD, H), jnp.float32),
            ]
            + [pltpu.SemaphoreType.DMA for _ in range(2 * NBUF + IBUF)]
        ),
    )
    def sc_agg(hs_hbm, src_hbm, dst_hbm, out_hbm, *refs):
        isrcr, idstr = refs[0:2]
        rows = refs[2:2 + NBUF]
        zer, acc = refs[2 + NBUF:4 + NBUF]
        gsem = refs[4 + NBUF:4 + 2 * NBUF]
        ssem = refs[4 + 2 * NBUF:4 + 3 * NBUF]
        isem = refs[4 + 3 * NBUF:]

        c = lax.axis_index("c")
        s = lax.axis_index("s")
        ebase = (c * NS + s) * CPT * CHUNK

        def zb(i, carry):
            for k in range(H // 16):
                zer[i, pl.ds(k * 16, 16)] = jnp.zeros((16,), jnp.float32)
            return carry

        lax.fori_loop(0, 32, zb, 0)

        def zc(i, carry):
            pltpu.sync_copy(zer, acc.at[pl.ds(s * ROWS_PT + i * 32, 32)])
            return carry

        lax.fori_loop(0, ROWS_PT // 32, zc, 0)
        plsc.subcore_barrier()

        def ifire(j, sl):
            base = ebase + j * CHUNK
            pltpu.async_copy(src_hbm.at[pl.ds(base, CHUNK)], isrcr.at[sl],
                             isem[sl])
            pltpu.async_copy(dst_hbm.at[pl.ds(base, CHUNK)], idstr.at[sl],
                             isem[sl])

        def idrain(j, sl):
            base = ebase + j * CHUNK
            pltpu.make_async_copy(src_hbm.at[pl.ds(base, CHUNK)],
                                  isrcr.at[sl], isem[sl]).wait()
            pltpu.make_async_copy(dst_hbm.at[pl.ds(base, CHUNK)],
                                  idstr.at[sl], isem[sl]).wait()

        def gfire(sl, b):
            pltpu.async_copy(hs_hbm.at[isrcr.at[sl]], rows[b], gsem[b])

        def gdrain(sl, b):
            pltpu.make_async_copy(hs_hbm.at[isrcr.at[sl]], rows[b],
                                  gsem[b]).wait()

        def sfire(sl, b):
            pltpu.async_copy(rows[b], acc.at[idstr.at[sl]], ssem[b],
                             add=True)

        def sdrain(sl, b):
            pltpu.make_async_copy(rows[b], acc.at[idstr.at[sl]],
                                  ssem[b]).wait()

        for sl in range(IBUF):
            ifire(sl, sl)
        for b in range(NBUF):
            idrain(b, b)
            gfire(b, b)

        def body(i, carry):
            j0 = i * IBUF
            for t in (0, 1):
                gdrain(t, t)
                sfire(t, t)
            for t in (0, 1):
                sdrain(t, t)
                ifire(j0 + 4 + t, t)
                idrain(j0 + 2 + t, t + 2)
                gfire(t + 2, t)
            for t in (2, 3):
                gdrain(t, t - 2)
                sfire(t, t - 2)
            for t in (2, 3):
                sdrain(t, t - 2)
                ifire(j0 + 4 + t, t)
                idrain(j0 + 4 + (t - 2), t - 2)
                gfire(t - 2, t - 2)
            return carry

        lax.fori_loop(0, CPT // IBUF - 1, body, 0)

        for t in (0, 1):
            gdrain(t, t)
            sfire(t, t)
        for t in (0, 1):
            sdrain(t, t)
            idrain(CPT - 2 + t, t + 2)
            gfire(t + 2, t)
        for t in (2, 3):
            gdrain(t, t - 2)
            sfire(t, t - 2)
        for t in (2, 3):
            sdrain(t, t - 2)

        plsc.subcore_barrier()
        pltpu.sync_copy(acc.at[pl.ds(s * ROWS_PT, ROWS_PT)],
                        out_hbm.at[c, pl.ds(s * ROWS_PT, ROWS_PT)])

    return sc_degree, sc_agg


_BLK = 2000
_GRID = N // _BLK


def _dinv_block(degp):
    deg = degp[:, 0] + degp[:, 1] + 1.0
    return lax.rsqrt(deg)


def _tc_in_body(x_ref, w_ref, degp_ref, h_ref, hs_ref):
    dinv = _dinv_block(degp_ref[...])
    h = jnp.dot(x_ref[...], w_ref[...], preferred_element_type=jnp.float32)
    h_ref[...] = h
    hs_ref[...] = h * dinv[:, None]


def _tc_in(x, W1, degp):
    return pl.pallas_call(
        _tc_in_body,
        grid=(_GRID,),
        in_specs=[
            pl.BlockSpec((_BLK, H), lambda i: (i, 0)),
            pl.BlockSpec((H, H), lambda i: (0, 0)),
            pl.BlockSpec((_BLK, NC), lambda i: (i, 0)),
        ],
        out_specs=[
            pl.BlockSpec((_BLK, H), lambda i: (i, 0)),
            pl.BlockSpec((_BLK, H), lambda i: (i, 0)),
        ],
        out_shape=[
            jax.ShapeDtypeStruct((N, H), jnp.float32),
            jax.ShapeDtypeStruct((N, H), jnp.float32),
        ],
    )(x, W1, degp)


def _tc_mid_body(aggp_ref, h1_ref, degp_ref, b1_ref, w2_ref, h2_ref, hs2_ref):
    dinv = _dinv_block(degp_ref[...])
    agg = aggp_ref[0, :, :] + aggp_ref[1, :, :]
    h1 = h1_ref[...]
    z = jnp.maximum(
        dinv[:, None] * agg + (dinv * dinv)[:, None] * h1
        + b1_ref[...][None, :], 0.0)
    h2 = jnp.dot(z, w2_ref[...], preferred_element_type=jnp.float32)
    h2_ref[...] = h2
    hs2_ref[...] = h2 * dinv[:, None]


def _tc_mid(aggp, h1, degp, b1, W2):
    return pl.pallas_call(
        _tc_mid_body,
        grid=(_GRID,),
        in_specs=[
            pl.BlockSpec((NC, _BLK, H), lambda i: (0, i, 0)),
            pl.BlockSpec((_BLK, H), lambda i: (i, 0)),
            pl.BlockSpec((_BLK, NC), lambda i: (i, 0)),
            pl.BlockSpec((H,), lambda i: (0,)),
            pl.BlockSpec((H, H), lambda i: (0, 0)),
        ],
        out_specs=[
            pl.BlockSpec((_BLK, H), lambda i: (i, 0)),
            pl.BlockSpec((_BLK, H), lambda i: (i, 0)),
        ],
        out_shape=[
            jax.ShapeDtypeStruct((N, H), jnp.float32),
            jax.ShapeDtypeStruct((N, H), jnp.float32),
        ],
    )(aggp, h1, degp, b1, W2)


def _tc_out_body(aggp_ref, h2_ref, degp_ref, b2_ref, batch_ref, wc_ref, bc_ref,
                 out_ref, pooled_acc, counts_acc):
    i = pl.program_id(0)
    dinv = _dinv_block(degp_ref[...])
    agg = aggp_ref[0, :, :] + aggp_ref[1, :, :]
    h2 = h2_ref[...]
    z = jnp.maximum(
        dinv[:, None] * agg + (dinv * dinv)[:, None] * h2
        + b2_ref[...][None, :], 0.0)
    gids = lax.broadcasted_iota(jnp.int32, (_BLK, G), 1)
    oh = (gids == batch_ref[...]).astype(jnp.float32)

    @pl.when(i == 0)
    def _():
        pooled_acc[...] = jnp.zeros((G, H), jnp.float32)
        counts_acc[...] = jnp.zeros((G, 1), jnp.float32)

    tn = (((0,), (0,)), ((), ()))
    pooled_acc[...] += lax.dot_general(oh, z, tn,
                                       preferred_element_type=jnp.float32)
    counts_acc[...] += lax.dot_general(oh, jnp.ones((_BLK, 1), jnp.float32),
                                       tn, preferred_element_type=jnp.float32)

    @pl.when(i == _GRID - 1)
    def _():
        pooled = pooled_acc[...] / jnp.maximum(counts_acc[...], 1.0)
        out_ref[...] = (jnp.dot(pooled, wc_ref[...],
                                preferred_element_type=jnp.float32)
                        + bc_ref[...][None, :])


def _tc_out(aggp, h2, degp, b2, batch, Wc, bc):
    return pl.pallas_call(
        _tc_out_body,
        grid=(_GRID,),
        in_specs=[
            pl.BlockSpec((NC, _BLK, H), lambda i: (0, i, 0)),
            pl.BlockSpec((_BLK, H), lambda i: (i, 0)),
            pl.BlockSpec((_BLK, NC), lambda i: (i, 0)),
            pl.BlockSpec((H,), lambda i: (0,)),
            pl.BlockSpec((_BLK, 1), lambda i: (i, 0)),
            pl.BlockSpec((H, C), lambda i: (0, 0)),
            pl.BlockSpec((C,), lambda i: (0,)),
        ],
        out_specs=pl.BlockSpec((G, C), lambda i: (0, 0)),
        out_shape=jax.ShapeDtypeStruct((G, C), jnp.float32),
        scratch_shapes=[
            pltpu.VMEM((G, H), jnp.float32),
            pltpu.VMEM((G, 1), jnp.float32),
        ],
    )(aggp, h2, degp, b2, batch, Wc, bc)


def _tile_chunks(v, pad):
    v2 = v.reshape(NCH, CHUNK)
    main = v2[:NW * (CPT - 2)].reshape(NW, CPT - 2, CHUNK)
    extra = v2[NW * (CPT - 2):]
    pad = pad.at[:extra.shape[0], 0].set(extra)
    return jnp.concatenate([main, pad], axis=1).reshape(-1)


def kernel(x, edge_index, batch, W1, b1, W2, b2, Wc, bc):
    sc_degree, sc_agg = _sc_kernels()
    seq = jnp.arange(NW * 2 * CHUNK, dtype=jnp.int32).reshape(NW, 2, CHUNK)
    src2 = _tile_chunks(edge_index[0], seq % N)
    dst2 = _tile_chunks(edge_index[1], N + seq % (NPAD - N))
    degp = jnp.transpose(sc_degree(dst2))
    batch2 = batch.reshape(N, 1)
    h1, hs1 = _tc_in(x, W1, degp)
    aggp1 = sc_agg(hs1, src2, dst2)
    h2, hs2 = _tc_mid(aggp1, h1, degp, b1, W2)
    aggp2 = sc_agg(hs2, src2, dst2)
    return _tc_out(aggp2, h2, degp, b2, batch2, Wc, bc)


# speedup vs baseline: 1.0074x; 1.0074x over previous
import functools

import jax
import jax.numpy as jnp
from jax import lax
from jax.experimental import pallas as pl
from jax.experimental.pallas import tpu as pltpu
from jax.experimental.pallas import tpu_sc as plsc

N = 10000
E = 320000
H = 128
G = 128
C = 10

NC = 2
NS = 16
NW = NC * NS
NPAD = 10240
ROWS_PT = NPAD // NS
CHUNK = 128
NCH = E // CHUNK
CPT = 80
NBUF = 2
IBUF = 4


@functools.cache
def _sc_kernels():
    mesh = plsc.VectorSubcoreMesh(core_axis_name="c", subcore_axis_name="s")

    @functools.partial(
        pl.kernel,
        mesh=mesh,
        out_type=jax.ShapeDtypeStruct((NC, NPAD), jnp.float32),
        scratch_types=[
            pltpu.VMEM((CPT * CHUNK,), jnp.int32),
            pltpu.VMEM((NBUF, CHUNK), jnp.int32),
            pltpu.VMEM((CHUNK,), jnp.float32),
            pltpu.VMEM((ROWS_PT,), jnp.float32),
            pltpu.VMEM_SHARED((NPAD,), jnp.float32),
        ] + [pltpu.SemaphoreType.DMA for _ in range(NBUF)],
    )
    def sc_degree(dst_hbm, out_hbm, idxall, idxr, ones_v, zero_v, acc, *ssem):
        c = lax.axis_index("c")
        s = lax.axis_index("s")
        w = c * NS + s

        pltpu.sync_copy(dst_hbm.at[pl.ds(w * CPT * CHUNK, CPT * CHUNK)],
                        idxall)

        for i in range(CHUNK // 16):
            ones_v[pl.ds(i * 16, 16)] = jnp.ones((16,), jnp.float32)

        def zb(i, carry):
            zero_v[pl.ds(i * 16, 16)] = jnp.zeros((16,), jnp.float32)
            return carry

        lax.fori_loop(0, ROWS_PT // 16, zb, 0)
        pltpu.sync_copy(zero_v, acc.at[pl.ds(s * ROWS_PT, ROWS_PT)])
        plsc.subcore_barrier()

        def fire_scatter(j, b):
            for k in range(CHUNK // 16):
                idxr[b, pl.ds(k * 16, 16)] = idxall[
                    pl.ds(j * CHUNK + k * 16, 16)]
            pltpu.async_copy(ones_v, acc.at[idxr.at[b]], ssem[b], add=True)

        for b in range(NBUF):
            fire_scatter(b, b)

        def body(i, carry):
            j0 = i * NBUF
            for b in range(NBUF):
                pltpu.make_async_copy(ones_v, acc.at[idxr.at[b]],
                                      ssem[b]).wait()
                fire_scatter(j0 + b + NBUF, b)
            return carry

        lax.fori_loop(0, CPT // NBUF - 1, body, 0)
        for b in range(NBUF):
            pltpu.make_async_copy(ones_v, acc.at[idxr.at[b]],
                                  ssem[b]).wait()

        plsc.subcore_barrier()
        pltpu.sync_copy(acc.at[pl.ds(s * ROWS_PT, ROWS_PT)],
                        out_hbm.at[c, pl.ds(s * ROWS_PT, ROWS_PT)])

    @functools.partial(
        pl.kernel,
        mesh=mesh,
        out_type=jax.ShapeDtypeStruct((NC, NPAD, H), jnp.float32),
        scratch_types=(
            [
                pltpu.VMEM((IBUF, CHUNK), jnp.int32),
                pltpu.VMEM((IBUF, CHUNK), jnp.int32),
            ]
            + [pltpu.VMEM((CHUNK, H), jnp.float32) for _ in range(NBUF)]
            + [
                pltpu.VMEM((32, H), jnp.float32),
                pltpu.VMEM_SHARED((NPAD, H), jnp.float32),
            ]
            + [pltpu.SemaphoreType.DMA for _ in range(2 * NBUF + IBUF)]
        ),
    )
    def sc_agg(hs_hbm, src_hbm, dst_hbm, out_hbm, *refs):
        isrcr, idstr = refs[0:2]
        rows = refs[2:2 + NBUF]
        zer, acc = refs[2 + NBUF:4 + NBUF]
        gsem = refs[4 + NBUF:4 + 2 * NBUF]
        ssem = refs[4 + 2 * NBUF:4 + 3 * NBUF]
        isem = refs[4 + 3 * NBUF:]

        c = lax.axis_index("c")
        s = lax.axis_index("s")
        ebase = (c * NS + s) * CPT * CHUNK

        def ifire(j, sl):
            base = ebase + j * CHUNK
            pltpu.async_copy(src_hbm.at[pl.ds(base, CHUNK)], isrcr.at[sl],
                             isem[sl])
            pltpu.async_copy(dst_hbm.at[pl.ds(base, CHUNK)], idstr.at[sl],
                             isem[sl])

        def idrain(j, sl):
            base = ebase + j * CHUNK
            pltpu.make_async_copy(src_hbm.at[pl.ds(base, CHUNK)],
                                  isrcr.at[sl], isem[sl]).wait()
            pltpu.make_async_copy(dst_hbm.at[pl.ds(base, CHUNK)],
                                  idstr.at[sl], isem[sl]).wait()

        def gfire(sl, b):
            pltpu.async_copy(hs_hbm.at[isrcr.at[sl]], rows[b], gsem[b])

        def gdrain(sl, b):
            pltpu.make_async_copy(hs_hbm.at[isrcr.at[sl]], rows[b],
                                  gsem[b]).wait()

        def sfire(sl, b):
            pltpu.async_copy(rows[b], acc.at[idstr.at[sl]], ssem[b],
                             add=True)

        def sdrain(sl, b):
            pltpu.make_async_copy(rows[b], acc.at[idstr.at[sl]],
                                  ssem[b]).wait()

        for sl in range(IBUF):
            ifire(sl, sl)
        for b in range(NBUF):
            idrain(b, b)
            gfire(b, b)

        def zb(i, carry):
            for k in range(H // 16):
                zer[i, pl.ds(k * 16, 16)] = jnp.zeros((16,), jnp.float32)
            return carry

        lax.fori_loop(0, 32, zb, 0)

        def zc(i, carry):
            pltpu.sync_copy(zer, acc.at[pl.ds(s * ROWS_PT + i * 32, 32)])
            return carry

        lax.fori_loop(0, ROWS_PT // 32, zc, 0)
        plsc.subcore_barrier()

        def body(i, carry):
            j0 = i * IBUF
            for t in (0, 1):
                gdrain(t, t)
                sfire(t, t)
            for t in (0, 1):
                sdrain(t, t)
                ifire(j0 + 4 + t, t)
                idrain(j0 + 2 + t, t + 2)
                gfire(t + 2, t)
            for t in (2, 3):
                gdrain(t, t - 2)
                sfire(t, t - 2)
            for t in (2, 3):
                sdrain(t, t - 2)
                ifire(j0 + 4 + t, t)
                idrain(j0 + 4 + (t - 2), t - 2)
                gfire(t - 2, t - 2)
            return carry

        lax.fori_loop(0, CPT // IBUF - 1, body, 0)

        for t in (0, 1):
            gdrain(t, t)
            sfire(t, t)
        for t in (0, 1):
            sdrain(t, t)
            idrain(CPT - 2 + t, t + 2)
            gfire(t + 2, t)
        for t in (2, 3):
            gdrain(t, t - 2)
            sfire(t, t - 2)
        for t in (2, 3):
            sdrain(t, t - 2)

        plsc.subcore_barrier()
        pltpu.sync_copy(acc.at[pl.ds(s * ROWS_PT, ROWS_PT)],
                        out_hbm.at[c, pl.ds(s * ROWS_PT, ROWS_PT)])

    return sc_degree, sc_agg


_BLK = 2000
_GRID = N // _BLK


def _dinv_block(degp):
    deg = degp[:, 0] + degp[:, 1] + 1.0
    return lax.rsqrt(deg)


def _tc_in_body(x_ref, w_ref, degp_ref, h_ref, hs_ref):
    dinv = _dinv_block(degp_ref[...])
    h = jnp.dot(x_ref[...], w_ref[...], preferred_element_type=jnp.float32)
    h_ref[...] = h
    hs_ref[...] = h * dinv[:, None]


def _tc_in(x, W1, degp):
    return pl.pallas_call(
        _tc_in_body,
        grid=(_GRID,),
        in_specs=[
            pl.BlockSpec((_BLK, H), lambda i: (i, 0)),
            pl.BlockSpec((H, H), lambda i: (0, 0)),
            pl.BlockSpec((_BLK, NC), lambda i: (i, 0)),
        ],
        out_specs=[
            pl.BlockSpec((_BLK, H), lambda i: (i, 0)),
            pl.BlockSpec((_BLK, H), lambda i: (i, 0)),
        ],
        out_shape=[
            jax.ShapeDtypeStruct((N, H), jnp.float32),
            jax.ShapeDtypeStruct((N, H), jnp.float32),
        ],
    )(x, W1, degp)


def _tc_mid_body(aggp_ref, h1_ref, degp_ref, b1_ref, w2_ref, h2_ref, hs2_ref):
    dinv = _dinv_block(degp_ref[...])
    agg = aggp_ref[0, :, :] + aggp_ref[1, :, :]
    h1 = h1_ref[...]
    z = jnp.maximum(
        dinv[:, None] * agg + (dinv * dinv)[:, None] * h1
        + b1_ref[...][None, :], 0.0)
    h2 = jnp.dot(z, w2_ref[...], preferred_element_type=jnp.float32)
    h2_ref[...] = h2
    hs2_ref[...] = h2 * dinv[:, None]


def _tc_mid(aggp, h1, degp, b1, W2):
    return pl.pallas_call(
        _tc_mid_body,
        grid=(_GRID,),
        in_specs=[
            pl.BlockSpec((NC, _BLK, H), lambda i: (0, i, 0)),
            pl.BlockSpec((_BLK, H), lambda i: (i, 0)),
            pl.BlockSpec((_BLK, NC), lambda i: (i, 0)),
            pl.BlockSpec((H,), lambda i: (0,)),
            pl.BlockSpec((H, H), lambda i: (0, 0)),
        ],
        out_specs=[
            pl.BlockSpec((_BLK, H), lambda i: (i, 0)),
            pl.BlockSpec((_BLK, H), lambda i: (i, 0)),
        ],
        out_shape=[
            jax.ShapeDtypeStruct((N, H), jnp.float32),
            jax.ShapeDtypeStruct((N, H), jnp.float32),
        ],
    )(aggp, h1, degp, b1, W2)


def _tc_out_body(aggp_ref, h2_ref, degp_ref, b2_ref, batch_ref, wc_ref, bc_ref,
                 out_ref, pooled_acc, counts_acc):
    i = pl.program_id(0)
    dinv = _dinv_block(degp_ref[...])
    agg = aggp_ref[0, :, :] + aggp_ref[1, :, :]
    h2 = h2_ref[...]
    z = jnp.maximum(
        dinv[:, None] * agg + (dinv * dinv)[:, None] * h2
        + b2_ref[...][None, :], 0.0)
    gids = lax.broadcasted_iota(jnp.int32, (_BLK, G), 1)
    oh = (gids == batch_ref[...]).astype(jnp.float32)

    @pl.when(i == 0)
    def _():
        pooled_acc[...] = jnp.zeros((G, H), jnp.float32)
        counts_acc[...] = jnp.zeros((G, 1), jnp.float32)

    tn = (((0,), (0,)), ((), ()))
    pooled_acc[...] += lax.dot_general(oh, z, tn,
                                       preferred_element_type=jnp.float32)
    counts_acc[...] += lax.dot_general(oh, jnp.ones((_BLK, 1), jnp.float32),
                                       tn, preferred_element_type=jnp.float32)

    @pl.when(i == _GRID - 1)
    def _():
        pooled = pooled_acc[...] / jnp.maximum(counts_acc[...], 1.0)
        out_ref[...] = (jnp.dot(pooled, wc_ref[...],
                                preferred_element_type=jnp.float32)
                        + bc_ref[...][None, :])


def _tc_out(aggp, h2, degp, b2, batch, Wc, bc):
    return pl.pallas_call(
        _tc_out_body,
        grid=(_GRID,),
        in_specs=[
            pl.BlockSpec((NC, _BLK, H), lambda i: (0, i, 0)),
            pl.BlockSpec((_BLK, H), lambda i: (i, 0)),
            pl.BlockSpec((_BLK, NC), lambda i: (i, 0)),
            pl.BlockSpec((H,), lambda i: (0,)),
            pl.BlockSpec((_BLK, 1), lambda i: (i, 0)),
            pl.BlockSpec((H, C), lambda i: (0, 0)),
            pl.BlockSpec((C,), lambda i: (0,)),
        ],
        out_specs=pl.BlockSpec((G, C), lambda i: (0, 0)),
        out_shape=jax.ShapeDtypeStruct((G, C), jnp.float32),
        scratch_shapes=[
            pltpu.VMEM((G, H), jnp.float32),
            pltpu.VMEM((G, 1), jnp.float32),
        ],
    )(aggp, h2, degp, b2, batch, Wc, bc)


def _tile_chunks(v, pad):
    v2 = v.reshape(NCH, CHUNK)
    main = v2[:NW * (CPT - 2)].reshape(NW, CPT - 2, CHUNK)
    extra = v2[NW * (CPT - 2):]
    pad = pad.at[:extra.shape[0], 0].set(extra)
    return jnp.concatenate([main, pad], axis=1).reshape(-1)


def kernel(x, edge_index, batch, W1, b1, W2, b2, Wc, bc):
    sc_degree, sc_agg = _sc_kernels()
    seq = jnp.arange(NW * 2 * CHUNK, dtype=jnp.int32).reshape(NW, 2, CHUNK)
    src2 = _tile_chunks(edge_index[0], seq % N)
    dst2 = _tile_chunks(edge_index[1], N + seq % (NPAD - N))
    degp = jnp.transpose(sc_degree(dst2))
    batch2 = batch.reshape(N, 1)
    h1, hs1 = _tc_in(x, W1, degp)
    aggp1 = sc_agg(hs1, src2, dst2)
    h2, hs2 = _tc_mid(aggp1, h1, degp, b1, W2)
    aggp2 = sc_agg(hs2, src2, dst2)
    return _tc_out(aggp2, h2, degp, b2, batch2, Wc, bc)
